# baseline (speedup 1.0000x reference)
import functools

import jax
import jax.numpy as jnp
from jax import lax
from jax.experimental import pallas as pl
from jax.experimental.pallas import tpu as pltpu
from jax.experimental.pallas import tpu_sc as plsc

VOCAB = 8192
BATCH = 4096
D = VOCAB

NUM_CORES = 2
NUM_SUBCORES = 16
NW = NUM_CORES * NUM_SUBCORES
B_PER_W = BATCH // NW
HD = D // 2
CHUNK = 8
N_CHUNKS = 2 * B_PER_W // CHUNK


def _gather_body(idx_hbm, table2_hbm, out3_hbm,
                 idx_v, idx2_v, buf_a, buf_b, gs_a, gs_b, ss_a, ss_b):
    wid = lax.axis_index("s") * NUM_CORES + lax.axis_index("c")
    base = wid * B_PER_W
    cid0 = wid * N_CHUNKS

    pltpu.sync_copy(idx_hbm.at[pl.ds(base, B_PER_W)], idx_v)

    lane = lax.iota(jnp.int32, 16)
    parity = lane & 1
    dnums = lax.GatherDimensionNumbers(
        offset_dims=(), collapsed_slice_dims=(0,), start_index_map=(0,))
    for g in range(B_PER_W // 16):
        v = idx_v[pl.ds(g * 16, 16)]
        for h in (0, 1):
            src = lax.gather(
                v, (h * 8 + (lane >> 1))[:, None],
                dimension_numbers=dnums, slice_sizes=(1,),
                mode=lax.GatherScatterMode.PROMISE_IN_BOUNDS)
            idx2_v[2 * g + h] = 2 * src + parity

    bufs = ((buf_a, gs_a, ss_a), (buf_b, gs_b, ss_b))

    def chunk_idx(row, half):
        return idx2_v.at[row, pl.ds(half * 8, CHUNK)]

    def start_gather(row, half, buf, gsem):
        pltpu.async_copy(table2_hbm.at[chunk_idx(row, half)], buf, gsem)

    def wait_gather(row, half, buf, gsem):
        pltpu.make_async_copy(
            table2_hbm.at[chunk_idx(row, half)], buf, gsem).wait()

    start_gather(0, 0, buf_a, gs_a)
    start_gather(0, 1, buf_b, gs_b)

    @pl.loop(0, N_CHUNKS - 2, step=2)
    def _step(k):
        row = k >> 1
        for b in (0, 1):
            buf, gsem, ssem = bufs[b]
            wait_gather(row, b, buf, gsem)
            pltpu.async_copy(buf, out3_hbm.at[cid0 + k + b], ssem).wait()
            start_gather(row + 1, b, buf, gsem)

    for b in (0, 1):
        kk = N_CHUNKS - 2 + b
        buf, gsem, ssem = bufs[b]
        wait_gather(kk >> 1, b, buf, gsem)
        pltpu.async_copy(buf, out3_hbm.at[cid0 + kk], ssem).wait()


@jax.jit
def _lookup(idx, table2):
    mesh = plsc.VectorSubcoreMesh(core_axis_name="c", subcore_axis_name="s")
    kfn = functools.partial(
        pl.kernel,
        mesh=mesh,
        out_type=jax.ShapeDtypeStruct((NW * N_CHUNKS, CHUNK, HD), jnp.float32),
        scratch_types=[
            pltpu.VMEM((B_PER_W,), jnp.int32),
            pltpu.VMEM((N_CHUNKS // 2, 16), jnp.int32),
            pltpu.VMEM((CHUNK, HD), jnp.float32),
            pltpu.VMEM((CHUNK, HD), jnp.float32),
            pltpu.SemaphoreType.DMA,
            pltpu.SemaphoreType.DMA,
            pltpu.SemaphoreType.DMA,
            pltpu.SemaphoreType.DMA,
        ],
    )(_gather_body)
    return kfn(idx, table2)


def kernel(x, table):
    last = x[:, -1].astype(jnp.int32)
    table2 = table.reshape(2 * VOCAB, HD)
    out3 = _lookup(last, table2)
    return out3.reshape(BATCH, D)


# speedup vs baseline: 2.0164x; 2.0164x over previous
import functools

import jax
import jax.numpy as jnp
from jax import lax
from jax.experimental import pallas as pl
from jax.experimental.pallas import tpu as pltpu
from jax.experimental.pallas import tpu_sc as plsc

VOCAB = 8192
BATCH = 4096
D = VOCAB

NUM_CORES = 2
NUM_SUBCORES = 16
NW = NUM_CORES * NUM_SUBCORES
B_PER_W = BATCH // NW
CHUNK = 4
N_CHUNKS = B_PER_W // CHUNK
NBUF = 2


def _gather_body(idx_hbm, table_hbm, out3_hbm, idx_v, idx_g, *bufs_and_sems):
    bufs = bufs_and_sems[:NBUF]
    gsems = bufs_and_sems[NBUF:2 * NBUF]
    ssems = bufs_and_sems[2 * NBUF:3 * NBUF]

    wid = lax.axis_index("s") * NUM_CORES + lax.axis_index("c")
    base = wid * B_PER_W
    cid0 = wid * N_CHUNKS

    pltpu.sync_copy(idx_hbm.at[pl.ds(base, B_PER_W)], idx_v)
    for r in range(B_PER_W // 16):
        idx_g[r] = idx_v[pl.ds(r * 16, 16)]

    def start_gather(k):
        idx_sl = idx_g.at[k // 4, pl.ds(CHUNK * (k % 4), CHUNK)]
        return pltpu.async_copy(table_hbm.at[idx_sl], bufs[k % NBUF],
                                gsems[k % NBUF])

    def start_scatter(k):
        return pltpu.async_copy(bufs[k % NBUF], out3_hbm.at[cid0 + k],
                                ssems[k % NBUF])

    gd = {}
    sd = {}
    for k in range(NBUF):
        gd[k] = start_gather(k)
    for k in range(N_CHUNKS):
        gd.pop(k).wait()
        sd[k] = start_scatter(k)
        if k + NBUF < N_CHUNKS:
            sd.pop(k).wait()
            gd[k + NBUF] = start_gather(k + NBUF)
    for k in range(N_CHUNKS - NBUF, N_CHUNKS):
        sd.pop(k).wait()


@jax.jit
def _lookup(idx, table):
    mesh = plsc.VectorSubcoreMesh(core_axis_name="c", subcore_axis_name="s")
    kfn = functools.partial(
        pl.kernel,
        mesh=mesh,
        out_type=jax.ShapeDtypeStruct((NW * N_CHUNKS, CHUNK, D), jnp.float32),
        scratch_types=(
            [pltpu.VMEM((B_PER_W,), jnp.int32),
             pltpu.VMEM((B_PER_W // 16, 16), jnp.int32)]
            + [pltpu.VMEM((CHUNK, D), jnp.float32)] * NBUF
            + [pltpu.SemaphoreType.DMA] * (2 * NBUF)
        ),
    )(_gather_body)
    return kfn(idx, table)


def kernel(x, table):
    last = x[:, -1].astype(jnp.int32)
    out3 = _lookup(last, table)
    return out3.reshape(BATCH, D)
